# baseline (speedup 1.0000x reference)
import functools

import jax
import jax.numpy as jnp
from jax import lax
from jax.experimental import pallas as pl
from jax.experimental.pallas import tpu as pltpu
from jax.experimental.pallas import tpu_sc as plsc

NC = 2
NS = 16
NW = NC * NS
LANES = 16
CHUNK_ROWS = 64


def kernel(x, pos_embedding):
    batch, seq_len, emb = x.shape
    x_flat = x.reshape(-1)
    pos_flat = pos_embedding[:seq_len].reshape(-1)

    seq_per_w = seq_len // NW
    n_chunks = seq_per_w // CHUNK_ROWS
    chunk_elems = CHUNK_ROWS * emb
    unroll = 8
    n_iters = chunk_elems // (LANES * unroll)

    mesh = plsc.VectorSubcoreMesh(core_axis_name="c", subcore_axis_name="s")

    @functools.partial(
        pl.kernel,
        mesh=mesh,
        out_type=jax.ShapeDtypeStruct((batch * seq_len * emb,), jnp.float32),
        scratch_types=[
            pltpu.VMEM((chunk_elems,), jnp.float32),
            pltpu.VMEM((chunk_elems,), jnp.float32),
        ],
    )
    def run(x_hbm, pos_hbm, out_hbm, x_v, pos_v):
        wid = lax.axis_index("s") * NC + lax.axis_index("c")
        seq_base = wid * seq_per_w

        def chunk_body(g, _):
            pbase = (seq_base + g * CHUNK_ROWS) * emb
            pltpu.sync_copy(pos_hbm.at[pl.ds(pbase, chunk_elems)], pos_v)
            for b in range(batch):
                xbase = (b * seq_len) * emb + pbase
                pltpu.sync_copy(x_hbm.at[pl.ds(xbase, chunk_elems)], x_v)

                def add_body(t, _):
                    base = t * (LANES * unroll)
                    for k in range(unroll):
                        o = base + k * LANES
                        x_v[pl.ds(o, LANES)] = (
                            x_v[pl.ds(o, LANES)] + pos_v[pl.ds(o, LANES)]
                        )
                    return 0

                lax.fori_loop(0, n_iters, add_body, 0)
                pltpu.sync_copy(x_v, out_hbm.at[pl.ds(xbase, chunk_elems)])
            return 0

        lax.fori_loop(0, n_chunks, chunk_body, 0)

    return run(x_flat, pos_flat).reshape(batch, seq_len, emb)


# speedup vs baseline: 1.1549x; 1.1549x over previous
import functools

import jax
import jax.numpy as jnp
from jax import lax
from jax.experimental import pallas as pl
from jax.experimental.pallas import tpu as pltpu
from jax.experimental.pallas import tpu_sc as plsc

NC = 2
NS = 16
NW = NC * NS
LANES = 16
CHUNK_ROWS = 32


def kernel(x, pos_embedding):
    batch, seq_len, emb = x.shape
    x_flat = x.reshape(-1)
    pos_flat = pos_embedding[:seq_len].reshape(-1)

    seq_per_w = seq_len // NW
    n_chunks = seq_per_w // CHUNK_ROWS
    chunk_elems = CHUNK_ROWS * emb
    unroll = 8
    n_iters = chunk_elems // (LANES * unroll)

    mesh = plsc.VectorSubcoreMesh(core_axis_name="c", subcore_axis_name="s")

    @functools.partial(
        pl.kernel,
        mesh=mesh,
        out_type=jax.ShapeDtypeStruct((batch * seq_len * emb,), jnp.float32),
        scratch_types=[
            pltpu.VMEM((chunk_elems,), jnp.float32),
            pltpu.VMEM((chunk_elems,), jnp.float32),
            pltpu.VMEM((chunk_elems,), jnp.float32),
            pltpu.VMEM((chunk_elems,), jnp.float32),
            pltpu.SemaphoreType.DMA,
            pltpu.SemaphoreType.DMA,
            pltpu.SemaphoreType.DMA,
            pltpu.SemaphoreType.DMA,
            pltpu.SemaphoreType.DMA,
            pltpu.SemaphoreType.DMA,
        ],
    )
    def run(x_hbm, pos_hbm, out_hbm, xv0, xv1, pv0, pv1,
            sin0, sin1, sout0, sout1, spos0, spos1):
        wid = lax.axis_index("s") * NC + lax.axis_index("c")
        seq_base = wid * seq_per_w

        xv = [xv0, xv1]
        sin = [sin0, sin1]
        sout = [sout0, sout1]
        pv = [pv0, pv1]
        spos = [spos0, spos1]

        steps = [(g, b) for g in range(n_chunks) for b in range(batch)]

        def pos_base(g):
            return (seq_base + g * CHUNK_ROWS) * emb

        def x_base(g, b):
            return (b * seq_len) * emb + pos_base(g)

        def start_pos(g):
            return pltpu.async_copy(
                pos_hbm.at[pl.ds(pos_base(g), chunk_elems)], pv[g % 2], spos[g % 2])

        def start_in(i):
            g, b = steps[i]
            return pltpu.async_copy(
                x_hbm.at[pl.ds(x_base(g, b), chunk_elems)], xv[i % 2], sin[i % 2])

        def start_out(i):
            g, b = steps[i]
            return pltpu.async_copy(
                xv[i % 2], out_hbm.at[pl.ds(x_base(g, b), chunk_elems)], sout[i % 2])

        pos_h = {0: start_pos(0)}
        in_h = {0: start_in(0)}
        out_h = {}

        for i, (g, b) in enumerate(steps):
            cur = i % 2
            if b == 0 and g + 1 < n_chunks:
                pos_h[g + 1] = start_pos(g + 1)
            if i + 1 < len(steps):
                if i >= 1:
                    out_h[i - 1].wait()
                in_h[i + 1] = start_in(i + 1)
            in_h[i].wait()
            if b == 0:
                pos_h[g].wait()

            buf = xv[cur]
            pbuf = pv[g % 2]

            def add_body(t, _, buf=buf, pbuf=pbuf):
                base = t * (LANES * unroll)
                for k in range(unroll):
                    o = base + k * LANES
                    buf[pl.ds(o, LANES)] = (
                        buf[pl.ds(o, LANES)] + pbuf[pl.ds(o, LANES)]
                    )
                return 0

            lax.fori_loop(0, n_iters, add_body, 0)
            out_h[i] = start_out(i)

        out_h[len(steps) - 2].wait()
        out_h[len(steps) - 1].wait()

    return run(x_flat, pos_flat).reshape(batch, seq_len, emb)
